# baseline (speedup 1.0000x reference)
import functools

import jax
import jax.numpy as jnp
from jax import lax
from jax.experimental import pallas as pl
from jax.experimental.pallas import tpu as pltpu
from jax.experimental.pallas import tpu_sc as plsc

BATCH = 16384
EMBED_DIM = 32
NUM_TOPICS = 1000000
NUM_CORES = 2
NUM_SUBCORES = 16
NUM_WORKERS = NUM_CORES * NUM_SUBCORES
B_PER_W = BATCH // NUM_WORKERS
CHUNK = 128
N_CHUNKS = B_PER_W // CHUNK

DEPAD_ROWS = 800
DEPAD_GRID = NUM_TOPICS // DEPAD_ROWS

_MESH = plsc.VectorSubcoreMesh(core_axis_name="c", subcore_axis_name="s")


def _depad_body(in_ref, out_ref):
    y = in_ref[...].reshape(DEPAD_ROWS // 4, 4, EMBED_DIM)
    for c in range(4):
        out_ref[:, 32 * c:32 * c + 32] = y[:, c, :]


_depad = pl.pallas_call(
    _depad_body,
    grid=(DEPAD_GRID,),
    in_specs=[pl.BlockSpec((DEPAD_ROWS, EMBED_DIM), lambda i: (i, 0))],
    out_specs=pl.BlockSpec((DEPAD_ROWS // 4, 128), lambda i: (i, 0)),
    out_shape=jax.ShapeDtypeStruct((NUM_TOPICS // 4, 128), jnp.float32),
)


@functools.partial(
    pl.kernel,
    mesh=_MESH,
    out_type=jax.ShapeDtypeStruct((BATCH, 128), jnp.float32),
    scratch_types=[
        pltpu.VMEM((N_CHUNKS, CHUNK), jnp.int32),
        pltpu.VMEM((CHUNK, 128), jnp.float32),
        pltpu.SemaphoreType.DMA,
    ],
    compiler_params=pltpu.CompilerParams(
        needs_layout_passes=False,
        disable_bounds_checks=True,
    ),
)
def _sc_gather(idx_hbm, table_hbm, out_hbm, idx_v, staged, sem):
    wid = lax.axis_index("s") * NUM_CORES + lax.axis_index("c")
    base = wid * B_PER_W
    pltpu.sync_copy(idx_hbm.at[pl.ds(wid * N_CHUNKS, N_CHUNKS)], idx_v)

    @pl.loop(0, N_CHUNKS)
    def _chunk(j):
        pltpu.async_copy(table_hbm.at[idx_v.at[j]], staged, sem).wait()
        pltpu.sync_copy(staged, out_hbm.at[pl.ds(base + j * CHUNK, CHUNK)])


def kernel(topic_labels, embedding_weight):
    idx = topic_labels.astype(jnp.int32)
    hi = (idx >> 2).reshape(BATCH // 128, 128)
    k = (idx & 3)[:, None]
    table2 = _depad(embedding_weight)
    out128 = _sc_gather(hi, table2)
    p0, p1, p2, p3 = (out128[:, 32 * c:32 * c + 32] for c in range(4))
    return jnp.where(
        k == 0, p0, jnp.where(k == 1, p1, jnp.where(k == 2, p2, p3))
    )


# speedup vs baseline: 3.6799x; 3.6799x over previous
import functools

import jax
import jax.numpy as jnp
from jax import lax
from jax.experimental import pallas as pl
from jax.experimental.pallas import tpu as pltpu
from jax.experimental.pallas import tpu_sc as plsc

BATCH = 16384
EMBED_DIM = 32
NUM_CORES = 2
NUM_SUBCORES = 16
NUM_WORKERS = NUM_CORES * NUM_SUBCORES
B_PER_W = BATCH // NUM_WORKERS
GROUP = 16
WINDOW = 64
N_WINDOWS = B_PER_W // WINDOW

_MESH = plsc.VectorSubcoreMesh(core_axis_name="c", subcore_axis_name="s")


@functools.partial(
    pl.kernel,
    mesh=_MESH,
    out_type=jax.ShapeDtypeStruct((BATCH, EMBED_DIM), jnp.float32),
    scratch_types=[
        pltpu.VMEM((B_PER_W,), jnp.int32),
        pltpu.VMEM((B_PER_W, EMBED_DIM), jnp.float32),
        pltpu.SemaphoreType.DMA,
        pltpu.SemaphoreType.DMA,
        pltpu.SemaphoreType.DMA,
        pltpu.SemaphoreType.DMA,
    ],
    compiler_params=pltpu.CompilerParams(needs_layout_passes=False),
)
def _sc_gather(idx_hbm, table_hbm, out_hbm, idx_v, rows_v, s0, s1, s2, s3):
    wid = lax.axis_index("s") * NUM_CORES + lax.axis_index("c")
    base = wid * B_PER_W
    sems = (s0, s1, s2, s3)
    pltpu.sync_copy(idx_hbm.at[wid], idx_v)

    @pl.loop(0, N_WINDOWS)
    def _win(g):
        copies = []
        for q in range(WINDOW // GROUP):
            iv = idx_v[pl.ds(g * WINDOW + q * GROUP, GROUP)]
            for t in range(GROUP):
                copies.append(
                    pltpu.async_copy(
                        table_hbm.at[iv[t]],
                        rows_v.at[g * WINDOW + q * GROUP + t],
                        sems[q],
                    )
                )
        for c in copies:
            c.wait()

    pltpu.sync_copy(rows_v, out_hbm.at[pl.ds(base, B_PER_W)])


def kernel(topic_labels, embedding_weight):
    idx = topic_labels.astype(jnp.int32).reshape(NUM_WORKERS, B_PER_W)
    return _sc_gather(idx, embedding_weight)
